# baseline (speedup 1.0000x reference)
import functools

import jax
import jax.numpy as jnp
from jax import lax
from jax.experimental import pallas as pl
from jax.experimental.pallas import tpu as pltpu
from jax.experimental.pallas import tpu_sc as plsc

D = 64
NC = 2
NS = 16
NW = NC * NS
C = 128
NB = 5


@functools.lru_cache(maxsize=None)
def _build(total):
    per_w = total // NW
    G = per_w // C
    assert per_w % C == 0 and G % NB == 0 and G > NB

    mesh = plsc.VectorSubcoreMesh(core_axis_name="c", subcore_axis_name="s")

    @functools.partial(
        pl.kernel,
        mesh=mesh,
        out_type=jax.ShapeDtypeStruct((total, D), jnp.float32),
        scratch_types=(
            [pltpu.VMEM((G, C), jnp.int32)]
            + [pltpu.VMEM((C, D), jnp.float32) for _ in range(NB)]
            + [pltpu.SemaphoreType.DMA for _ in range(NB)]
        ),
        compiler_params=pltpu.CompilerParams(use_tc_tiling_on_sc=False),
    )
    def k(table_hbm, idx_hbm, out_hbm, idx_v, *rest):
        bufs = rest[:NB]
        sems = rest[NB:]
        wid = lax.axis_index("s") * NC + lax.axis_index("c")
        base = wid * per_w

        pltpu.sync_copy(idx_hbm.at[wid], idx_v)

        for b in range(NB):
            pltpu.async_copy(table_hbm.at[idx_v.at[b]], bufs[b], sems[b])

        def outer(o, carry):
            for b in range(NB):
                g = o * NB + b
                pltpu.make_async_copy(
                    table_hbm.at[idx_v.at[g]], bufs[b], sems[b]
                ).wait()
                pltpu.sync_copy(bufs[b], out_hbm.at[pl.ds(base + g * C, C)])
                pltpu.async_copy(
                    table_hbm.at[idx_v.at[g + NB]], bufs[b], sems[b]
                )
            return carry

        lax.fori_loop(0, (G - NB) // NB, outer, 0)

        for b in range(NB):
            g = G - NB + b
            pltpu.make_async_copy(
                table_hbm.at[idx_v.at[g]], bufs[b], sems[b]
            ).wait()
            pltpu.sync_copy(bufs[b], out_hbm.at[pl.ds(base + g * C, C)])

    return k


def kernel(inputs, embedding):
    bsz, hist = inputs.shape
    total = bsz * hist
    idx = inputs.reshape(NW, total // (NW * C), C).astype(jnp.int32)
    out = _build(total)(embedding, idx)
    return out.reshape(bsz, hist, D)
